# baseline (speedup 1.0000x reference)
import functools

import jax
import jax.numpy as jnp
from jax import lax
from jax.experimental import pallas as pl
from jax.experimental.pallas import tpu as pltpu
from jax.experimental.pallas import tpu_sc as plsc

_NC = 2
_NS = 16
_L = 16
_ROWS = 256
_NBUF = 4


def _build_adjacency(data, data2, n):
    e = data.shape[0]
    ec = e // _NS
    passes = n // (_ROWS * _NC)
    wpt = (_ROWS * n) // _NS
    gran = _NBUF * 128
    cap = 2 * ec + gran + _L
    garbage = _ROWS * n

    mesh = plsc.VectorSubcoreMesh(core_axis_name="c", subcore_axis_name="s")

    scratch = [
        pltpu.VMEM((ec,), jnp.int32),
        pltpu.VMEM((ec,), jnp.int32),
        pltpu.VMEM((1024,), jnp.float32),
        pltpu.VMEM((16384,), jnp.float32),
    ]
    scratch += [pltpu.VMEM((1024,), jnp.int32) for _ in range(2)]
    scratch += [
        pltpu.VMEM_SHARED((_ROWS * n + _L,), jnp.float32),
        pltpu.SemaphoreType.DMA,
    ]

    @functools.partial(
        pl.kernel,
        out_type=jax.ShapeDtypeStruct((n * n,), jnp.float32),
        mesh=mesh,
        scratch_types=scratch,
    )
    def build(d1_hbm, d2_hbm, out_hbm, ei_v, ej_v, ones_v, zero_v, *rest):
        scat = rest[:2]
        shared, sem = rest[2:]
        c = lax.axis_index("c")
        s = lax.axis_index("s")

        pltpu.sync_copy(d1_hbm.at[pl.ds(s * ec, ec)], ei_v)
        pltpu.sync_copy(d2_hbm.at[pl.ds(s * ec, ec)], ej_v)

        zeros16 = jnp.zeros((_L,), jnp.float32)
        ones16 = jnp.ones((_L,), jnp.float32)

        def zfill(i, carry):
            zero_v[pl.ds(i * _L, _L)] = zeros16
            return carry
        lax.fori_loop(0, 16384 // _L, zfill, 0)
        for q in range(1024 // _L):
            ones_v[pl.ds(q * _L, _L)] = ones16

        def do_pass(p, carry):
            base = (p * _NC + c) * _ROWS

            def zcp(i, cz):
                pltpu.sync_copy(
                    zero_v, shared.at[pl.ds(s * wpt + i * 16384, 16384)])
                return cz
            lax.fori_loop(0, wpt // 16384, zcp, 0)
            plsc.subcore_barrier()

            def bp_loop(bp, cb):
                ebase = bp * 1024

                def fill_block(ebase2, rr):
                    for r in range(8):
                        def fillq(q, cf, r=r, rr=rr):
                            off = ebase2 + (r * 4 + q) * _L
                            vi = ei_v[pl.ds(off, _L)]
                            vj = ej_v[pl.ds(off, _L)]
                            for pair in range(2):
                                row = vi if pair == 0 else vj
                                col = vj if pair == 0 else vi
                                rel = row - base
                                m = (rel >= 0) & (rel < _ROWS)
                                idx = jnp.where(m, rel * n + col, garbage)
                                scat[rr][pl.ds((r * 4 + q) * 2 * _L
                                               + pair * _L, _L)] = idx
                            return cf
                        lax.fori_loop(0, 4, fillq, 0)
                    return pltpu.async_copy(
                        ones_v, shared.at[scat[rr]], sem, add=True)

                d0 = fill_block(ebase, 0)
                d1 = fill_block(ebase + 512, 1)
                d0.wait()
                d1.wait()
                return cb
            lax.fori_loop(0, ec // 1024, bp_loop, 0)
            plsc.subcore_barrier()

            pltpu.sync_copy(
                shared.at[pl.ds(s * wpt, wpt)],
                out_hbm.at[pl.ds(base * n + s * wpt, wpt)])
            return carry
        lax.fori_loop(0, passes, do_pass, 0)

    return build(data, data2)


def _factors(s_mat, c_mat, a16):
    n = c_mat.shape[0]
    k = c_mat.shape[1]
    blk = 512

    def rs_body(a_ref, rdeg_ref):
        rs = jnp.sum(a_ref[...].astype(jnp.float32), axis=1)
        rdeg_ref[...] = (0.5 / rs)[None, :]

    rdeg = pl.pallas_call(
        rs_body,
        grid=(n // blk,),
        in_specs=[pl.BlockSpec((blk, n), lambda i: (i, 0))],
        out_specs=pl.BlockSpec((1, blk), lambda i: (0, i)),
        out_shape=jax.ShapeDtypeStruct((1, n), jnp.float32),
    )(a16)

    def sm_body(s_ref, c_ref, csm_ref, ssm_ref, r2_ref):
        ssm = jax.nn.softmax(s_ref[...], axis=0)
        csm = jax.nn.softmax(c_ref[...], axis=0)
        ssm_ref[...] = ssm
        csm_ref[...] = csm
        g = lax.dot_general(ssm, ssm, (((1,), (1,)), ((), ())),
                            preferred_element_type=jnp.float32)
        r2_ref[...] = jnp.dot(csm, g, preferred_element_type=jnp.float32)

    csm, ssm, r2 = pl.pallas_call(
        sm_body,
        out_shape=[
            jax.ShapeDtypeStruct((n, k), jnp.float32),
            jax.ShapeDtypeStruct((k, n), jnp.float32),
            jax.ShapeDtypeStruct((n, k), jnp.float32),
        ],
    )(s_mat, c_mat)
    return rdeg, csm, ssm, r2


def _contract(a16, csm, ssm, r2, rdeg):
    n = a16.shape[0]
    k = csm.shape[1]
    blk = 512
    g = n // blk

    def body(ai, aj, ci, sj, rj, di, dj, out):
        i = pl.program_id(0)
        j = pl.program_id(1)

        @pl.when((i == 0) & (j == 0))
        def _init():
            out[...] = jnp.zeros((1, 1), jnp.float32)

        t = lax.dot_general(ai[...], aj[...], (((1,), (1,)), ((), ())),
                            preferred_element_type=jnp.float32)
        q1 = lax.dot_general(ci[...], rj[...], (((1,), (1,)), ((), ())),
                             preferred_element_type=jnp.float32)
        q2 = lax.dot_general(ci[...], sj[...], (((1,), (0,)), ((), ())),
                             preferred_element_type=jnp.float32)
        q = q1 - 2.0 * q2
        ri = di[0, :]
        rjv = dj[0, :]
        km = t * ri[:, None] * rjv[None, :]
        ind = (t > 0.0).astype(jnp.float32)
        out[...] += jnp.sum((ind - km) * q, axis=(0, 1), keepdims=True)

    return pl.pallas_call(
        body,
        grid=(g, g),
        in_specs=[
            pl.BlockSpec((blk, n), lambda i, j: (i, 0)),
            pl.BlockSpec((blk, n), lambda i, j: (j, 0)),
            pl.BlockSpec((blk, k), lambda i, j: (i, 0)),
            pl.BlockSpec((k, blk), lambda i, j: (0, j)),
            pl.BlockSpec((blk, k), lambda i, j: (j, 0)),
            pl.BlockSpec((1, blk), lambda i, j: (0, i)),
            pl.BlockSpec((1, blk), lambda i, j: (0, j)),
        ],
        out_specs=pl.BlockSpec((1, 1), lambda i, j: (0, 0)),
        out_shape=jax.ShapeDtypeStruct((1, 1), jnp.float32),
        compiler_params=pltpu.CompilerParams(
            dimension_semantics=("arbitrary", "arbitrary")),
    )(a16, a16, csm, ssm, r2, rdeg, rdeg)


def kernel(S, C, a, data, data2, non_sparse_i, non_sparse_j,
           sparse_i_rem, sparse_j_rem):
    n = S.shape[1]
    a_flat = _build_adjacency(data, data2, n)
    a16 = a_flat.reshape(n, n).astype(jnp.bfloat16)
    rdeg, csm, ssm, r2 = _factors(S, C, a16)
    out = _contract(a16, csm, ssm, r2, rdeg)
    return out[0, 0]


# speedup vs baseline: 1.0012x; 1.0012x over previous
import functools

import jax
import jax.numpy as jnp
from jax import lax
from jax.experimental import pallas as pl
from jax.experimental.pallas import tpu as pltpu
from jax.experimental.pallas import tpu_sc as plsc

_NC = 2
_NS = 16
_L = 16
_ROWS = 256
_NBUF = 4


def _build_adjacency(data, data2, n):
    e = data.shape[0]
    ec = e // _NS
    passes = n // (_ROWS * _NC)
    wpt = (_ROWS * n) // _NS
    gran = _NBUF * 128
    cap = 2 * ec + gran + _L
    garbage = _ROWS * n

    mesh = plsc.VectorSubcoreMesh(core_axis_name="c", subcore_axis_name="s")

    scratch = [
        pltpu.VMEM((ec,), jnp.int32),
        pltpu.VMEM((ec,), jnp.int32),
        pltpu.VMEM((1024,), jnp.float32),
        pltpu.VMEM((16384,), jnp.float32),
        pltpu.VMEM((1024,), jnp.int32),
        pltpu.VMEM((1024,), jnp.int32),
        pltpu.VMEM_SHARED((_ROWS * n + _L,), jnp.float32),
        pltpu.SemaphoreType.DMA,
    ]

    @functools.partial(
        pl.kernel,
        out_type=jax.ShapeDtypeStruct((n * n,), jnp.float32),
        mesh=mesh,
        scratch_types=scratch,
    )
    def build(d1_hbm, d2_hbm, out_hbm, ei_v, ej_v, ones_v, zero_v,
              scat_a, scat_b, shared, sem):
        scat = (scat_a, scat_b)
        c = lax.axis_index("c")
        s = lax.axis_index("s")

        pltpu.sync_copy(d1_hbm.at[pl.ds(s * ec, ec)], ei_v)
        pltpu.sync_copy(d2_hbm.at[pl.ds(s * ec, ec)], ej_v)

        zeros16 = jnp.zeros((_L,), jnp.float32)
        ones16 = jnp.ones((_L,), jnp.float32)

        def zfill(i, carry):
            zero_v[pl.ds(i * _L, _L)] = zeros16
            return carry
        lax.fori_loop(0, 16384 // _L, zfill, 0)
        for q in range(1024 // _L):
            ones_v[pl.ds(q * _L, _L)] = ones16

        nblocks = ec // 512

        def do_pass(p, carry):
            base = (p * _NC + c) * _ROWS

            def zcp(i, cz):
                pltpu.sync_copy(
                    zero_v, shared.at[pl.ds(s * wpt + i * 16384, 16384)])
                return cz
            lax.fori_loop(0, wpt // 16384, zcp, 0)
            plsc.subcore_barrier()

            def fill_block(ebase, rr):
                for r in range(8):
                    def fillq(q, cf, r=r, rr=rr):
                        off = ebase + (r * 4 + q) * _L
                        vi = ei_v[pl.ds(off, _L)]
                        vj = ej_v[pl.ds(off, _L)]
                        for pair in range(2):
                            row = vi if pair == 0 else vj
                            col = vj if pair == 0 else vi
                            rel = row - base
                            m = (rel >= 0) & (rel < _ROWS)
                            idx = jnp.where(m, rel * n + col, garbage)
                            scat[rr][pl.ds((r * 4 + q) * 2 * _L
                                           + pair * _L, _L)] = idx
                        return cf
                    lax.fori_loop(0, 4, fillq, 0)

            def fire(rr):
                return pltpu.async_copy(
                    ones_v, shared.at[scat[rr]], sem, add=True)

            def drain(rr):
                pltpu.make_async_copy(
                    ones_v, shared.at[scat[rr]], sem).wait()

            fill_block(0, 0)
            fire(0)

            def bp_loop(g, cb):
                ebase = g * 512
                fill_block(ebase, 1)
                drain(0)
                fire(1)
                return cb

            def bp2(h, cb):
                fill_block((2 * h + 1) * 512, 1)
                drain(0)
                fire(1)
                fill_block((2 * h + 2) * 512, 0)
                drain(1)
                fire(0)
                return cb
            lax.fori_loop(0, (nblocks - 2) // 2, bp2, 0)
            fill_block((nblocks - 1) * 512, 1)
            drain(0)
            fire(1)
            drain(1)
            plsc.subcore_barrier()

            pltpu.sync_copy(
                shared.at[pl.ds(s * wpt, wpt)],
                out_hbm.at[pl.ds(base * n + s * wpt, wpt)])
            return carry
        lax.fori_loop(0, passes, do_pass, 0)

    return build(data, data2)


def _factors(s_mat, c_mat, a16):
    n = c_mat.shape[0]
    k = c_mat.shape[1]
    blk = 512

    def rs_body(a_ref, rdeg_ref):
        rs = jnp.sum(a_ref[...].astype(jnp.float32), axis=1)
        rdeg_ref[...] = (0.5 / rs)[None, :]

    rdeg = pl.pallas_call(
        rs_body,
        grid=(n // blk,),
        in_specs=[pl.BlockSpec((blk, n), lambda i: (i, 0))],
        out_specs=pl.BlockSpec((1, blk), lambda i: (0, i)),
        out_shape=jax.ShapeDtypeStruct((1, n), jnp.float32),
    )(a16)

    def sm_body(s_ref, c_ref, csm_ref, ssm_ref, r2_ref):
        ssm = jax.nn.softmax(s_ref[...], axis=0)
        csm = jax.nn.softmax(c_ref[...], axis=0)
        ssm_ref[...] = ssm
        csm_ref[...] = csm
        g = lax.dot_general(ssm, ssm, (((1,), (1,)), ((), ())),
                            preferred_element_type=jnp.float32)
        r2_ref[...] = jnp.dot(csm, g, preferred_element_type=jnp.float32)

    csm, ssm, r2 = pl.pallas_call(
        sm_body,
        out_shape=[
            jax.ShapeDtypeStruct((n, k), jnp.float32),
            jax.ShapeDtypeStruct((k, n), jnp.float32),
            jax.ShapeDtypeStruct((n, k), jnp.float32),
        ],
    )(s_mat, c_mat)
    return rdeg, csm, ssm, r2


def _contract(a16, csm, ssm, r2, rdeg):
    n = a16.shape[0]
    k = csm.shape[1]
    blk = 512
    g = n // blk

    def body(ai, aj, ci, sj, rj, di, dj, out):
        i = pl.program_id(0)
        j = pl.program_id(1)

        @pl.when((i == 0) & (j == 0))
        def _init():
            out[...] = jnp.zeros((1, 1), jnp.float32)

        t = lax.dot_general(ai[...], aj[...], (((1,), (1,)), ((), ())),
                            preferred_element_type=jnp.float32)
        q1 = lax.dot_general(ci[...], rj[...], (((1,), (1,)), ((), ())),
                             preferred_element_type=jnp.float32)
        q2 = lax.dot_general(ci[...], sj[...], (((1,), (0,)), ((), ())),
                             preferred_element_type=jnp.float32)
        q = q1 - 2.0 * q2
        ri = di[0, :]
        rjv = dj[0, :]
        km = t * ri[:, None] * rjv[None, :]
        ind = (t > 0.0).astype(jnp.float32)
        out[...] += jnp.sum((ind - km) * q, axis=(0, 1), keepdims=True)

    return pl.pallas_call(
        body,
        grid=(g, g),
        in_specs=[
            pl.BlockSpec((blk, n), lambda i, j: (i, 0)),
            pl.BlockSpec((blk, n), lambda i, j: (j, 0)),
            pl.BlockSpec((blk, k), lambda i, j: (i, 0)),
            pl.BlockSpec((k, blk), lambda i, j: (0, j)),
            pl.BlockSpec((blk, k), lambda i, j: (j, 0)),
            pl.BlockSpec((1, blk), lambda i, j: (0, i)),
            pl.BlockSpec((1, blk), lambda i, j: (0, j)),
        ],
        out_specs=pl.BlockSpec((1, 1), lambda i, j: (0, 0)),
        out_shape=jax.ShapeDtypeStruct((1, 1), jnp.float32),
        compiler_params=pltpu.CompilerParams(
            dimension_semantics=("arbitrary", "arbitrary")),
    )(a16, a16, csm, ssm, r2, rdeg, rdeg)


def kernel(S, C, a, data, data2, non_sparse_i, non_sparse_j,
           sparse_i_rem, sparse_j_rem):
    n = S.shape[1]
    a_flat = _build_adjacency(data, data2, n)
    a16 = a_flat.reshape(n, n).astype(jnp.bfloat16)
    rdeg, csm, ssm, r2 = _factors(S, C, a16)
    out = _contract(a16, csm, ssm, r2, rdeg)
    return out[0, 0]


# speedup vs baseline: 1.6483x; 1.6464x over previous
import functools

import jax
import jax.numpy as jnp
from jax import lax
from jax.experimental import pallas as pl
from jax.experimental.pallas import tpu as pltpu
from jax.experimental.pallas import tpu_sc as plsc

_NC = 2
_NS = 16
_L = 16
_ROWS = 256
_NBUF = 4


def _build_adjacency(data, data2, n):
    e = data.shape[0]
    ec = e // _NS
    passes = n // (_ROWS * _NC)
    wpt = (_ROWS * n) // _NS
    garbage = _ROWS * n

    mesh = plsc.VectorSubcoreMesh(core_axis_name="c", subcore_axis_name="s")

    scratch = [
        pltpu.VMEM((ec,), jnp.int32),
        pltpu.VMEM((ec,), jnp.int32),
        pltpu.VMEM((1024,), jnp.float32),
        pltpu.VMEM((16384,), jnp.float32),
        pltpu.VMEM((1024,), jnp.int32),
        pltpu.VMEM((1024,), jnp.int32),
        pltpu.VMEM_SHARED((_ROWS * n + 64,), jnp.float32),
        pltpu.SemaphoreType.DMA,
    ]

    @functools.partial(
        pl.kernel,
        out_type=jax.ShapeDtypeStruct((n * n,), jnp.float32),
        mesh=mesh,
        scratch_types=scratch,
    )
    def build(d1_hbm, d2_hbm, ones_hbm, zeros_hbm, out_hbm, ei_v, ej_v,
              ones_v, zero_v, scat_a, scat_b, shared, sem):
        scat = (scat_a, scat_b)
        c = lax.axis_index("c")
        s = lax.axis_index("s")

        pltpu.sync_copy(d1_hbm.at[pl.ds(s * ec, ec)], ei_v)
        pltpu.sync_copy(d2_hbm.at[pl.ds(s * ec, ec)], ej_v)
        pltpu.sync_copy(ones_hbm, ones_v)
        pltpu.sync_copy(zeros_hbm, zero_v)

        nblocks = ec // 1024

        def do_pass(p, carry):
            base = (p * _NC + c) * _ROWS

            def zcp(i, cz):
                pltpu.sync_copy(
                    zero_v, shared.at[pl.ds(s * wpt + i * 16384, 16384)])
                return cz
            lax.fori_loop(0, wpt // 16384, zcp, 0)
            plsc.subcore_barrier()

            def fill_block(ebase, rr):
                for r in range(8):
                    def fillq(q, cf, r=r, rr=rr):
                        off = ebase + (r * 8 + q) * _L
                        vi = ei_v[pl.ds(off, _L)]
                        vj = ej_v[pl.ds(off, _L)]
                        row = jnp.minimum(vi, vj)
                        col = jnp.maximum(vi, vj)
                        rel = row - base
                        m = (rel >= 0) & (rel < _ROWS)
                        idx = jnp.where(m, rel * n + col, garbage)
                        scat[rr][pl.ds((r * 8 + q) * _L, _L)] = idx
                        return cf
                    lax.fori_loop(0, 8, fillq, 0)

            def fire(rr):
                return pltpu.async_copy(
                    ones_v, shared.at[scat[rr]], sem, add=True)

            def drain(rr):
                pltpu.make_async_copy(
                    ones_v, shared.at[scat[rr]], sem).wait()

            fill_block(0, 0)
            fire(0)

            def bp2(h, cb):
                fill_block((2 * h + 1) * 1024, 1)
                drain(0)
                fire(1)
                fill_block((2 * h + 2) * 1024, 0)
                drain(1)
                fire(0)
                return cb
            lax.fori_loop(0, (nblocks - 2) // 2, bp2, 0)
            fill_block((nblocks - 1) * 1024, 1)
            drain(0)
            fire(1)
            drain(1)
            plsc.subcore_barrier()

            pltpu.sync_copy(
                shared.at[pl.ds(s * wpt, wpt)],
                out_hbm.at[pl.ds(base * n + s * wpt, wpt)])
            return carry
        lax.fori_loop(0, passes, do_pass, 0)

    ones_in = jnp.ones((1024,), jnp.float32)
    zeros_in = jnp.zeros((16384,), jnp.float32)
    return build(data, data2, ones_in, zeros_in)


def _factors(s_mat, c_mat, a16):
    n = c_mat.shape[0]
    k = c_mat.shape[1]
    blk = 512

    def rs_body(a_ref, rdeg_ref):
        rs = jnp.sum(a_ref[...].astype(jnp.float32), axis=1)
        rdeg_ref[...] = (0.5 / rs)[None, :]

    rdeg = pl.pallas_call(
        rs_body,
        grid=(n // blk,),
        in_specs=[pl.BlockSpec((blk, n), lambda i: (i, 0))],
        out_specs=pl.BlockSpec((1, blk), lambda i: (0, i)),
        out_shape=jax.ShapeDtypeStruct((1, n), jnp.float32),
    )(a16)

    def sm_body(s_ref, c_ref, csm_ref, ssm_ref, r2_ref):
        ssm = jax.nn.softmax(s_ref[...], axis=0)
        csm = jax.nn.softmax(c_ref[...], axis=0)
        ssm_ref[...] = ssm
        csm_ref[...] = csm
        g = lax.dot_general(ssm, ssm, (((1,), (1,)), ((), ())),
                            preferred_element_type=jnp.float32)
        r2_ref[...] = jnp.dot(csm, g, preferred_element_type=jnp.float32)

    csm, ssm, r2 = pl.pallas_call(
        sm_body,
        out_shape=[
            jax.ShapeDtypeStruct((n, k), jnp.float32),
            jax.ShapeDtypeStruct((k, n), jnp.float32),
            jax.ShapeDtypeStruct((n, k), jnp.float32),
        ],
    )(s_mat, c_mat)
    return rdeg, csm, ssm, r2


def _contract(a16, csm, ssm, r2, rdeg):
    n = a16.shape[0]
    k = csm.shape[1]
    blk = 512
    g = n // blk

    def body(ai, aj, ci, sj, rj, di, dj, out):
        i = pl.program_id(0)
        j = pl.program_id(1)

        @pl.when((i == 0) & (j == 0))
        def _init():
            out[...] = jnp.zeros((1, 1), jnp.float32)

        t = lax.dot_general(ai[...], aj[...], (((1,), (1,)), ((), ())),
                            preferred_element_type=jnp.float32)
        q1 = lax.dot_general(ci[...], rj[...], (((1,), (1,)), ((), ())),
                             preferred_element_type=jnp.float32)
        q2 = lax.dot_general(ci[...], sj[...], (((1,), (0,)), ((), ())),
                             preferred_element_type=jnp.float32)
        q = q1 - 2.0 * q2
        ri = di[0, :]
        rjv = dj[0, :]
        km = t * ri[:, None] * rjv[None, :]
        ind = (t > 0.0).astype(jnp.float32)
        out[...] += jnp.sum((ind - km) * q, axis=(0, 1), keepdims=True)

    return pl.pallas_call(
        body,
        grid=(g, g),
        in_specs=[
            pl.BlockSpec((blk, n), lambda i, j: (i, 0)),
            pl.BlockSpec((blk, n), lambda i, j: (j, 0)),
            pl.BlockSpec((blk, k), lambda i, j: (i, 0)),
            pl.BlockSpec((k, blk), lambda i, j: (0, j)),
            pl.BlockSpec((blk, k), lambda i, j: (j, 0)),
            pl.BlockSpec((1, blk), lambda i, j: (0, i)),
            pl.BlockSpec((1, blk), lambda i, j: (0, j)),
        ],
        out_specs=pl.BlockSpec((1, 1), lambda i, j: (0, 0)),
        out_shape=jax.ShapeDtypeStruct((1, 1), jnp.float32),
        compiler_params=pltpu.CompilerParams(
            dimension_semantics=("arbitrary", "arbitrary")),
    )(a16, a16, csm, ssm, r2, rdeg, rdeg)


def kernel(S, C, a, data, data2, non_sparse_i, non_sparse_j,
           sparse_i_rem, sparse_j_rem):
    n = S.shape[1]
    u = _build_adjacency(data, data2, n).reshape(n, n)
    a16 = (u + u.T).astype(jnp.bfloat16)
    rdeg, csm, ssm, r2 = _factors(S, C, a16)
    out = _contract(a16, csm, ssm, r2, rdeg)
    return out[0, 0]
